# Initial kernel scaffold; baseline (speedup 1.0000x reference)
import jax
import jax.numpy as jnp
from jax.experimental import pallas as pl


def kernel(x, lookup_values):
    raise NotImplementedError("write your pallas kernel here")


# speedup vs baseline: 3.1312x; 3.1312x over previous
import dataclasses
import functools

import jax
import jax.numpy as jnp
from jax import lax
from jax.experimental import pallas as pl
from jax.experimental.pallas import tpu as pltpu
from jax.experimental.pallas import tpu_sc as plsc

GS = 128
NLEV = 16
L = 16
BLOCK = 8192
INV_MAXQ2 = 2.0 / 15.0


def kernel(x, lookup_values):
    shape = x.shape
    n = x.size
    x1 = x.reshape(n)
    mesh = plsc.VectorSubcoreMesh(core_axis_name="c", subcore_axis_name="s")
    cp = pltpu.CompilerParams()
    if "needs_layout_passes" in pltpu.CompilerParams.__dataclass_fields__:
        cp = dataclasses.replace(cp, needs_layout_passes=False)

    @functools.partial(
        pl.kernel,
        mesh=mesh,
        out_type=jax.ShapeDtypeStruct((n,), jnp.float32),
        scratch_types=[pltpu.VMEM((NLEV,), jnp.float32)],
        compiler_params=cp,
    )
    def run(x_hbm, lut_hbm, o_hbm, lut_vmem):
        pltpu.sync_copy(lut_hbm, lut_vmem)
        lut = lut_vmem[...]
        cbv = [jnp.full((L,), lut[i], jnp.float32) for i in range(NLEV)]
        midv = [(cbv[i] + cbv[i + 1]) * 0.5 for i in range(NLEV - 1)]

        def nearest(q):
            r = cbv[0]
            for k in range(1, NLEV):
                r = jnp.where(q > midv[k - 1], cbv[k], r)
            return r

        zv = nearest(jnp.full((L,), 8.0, jnp.float32))

        def body(x_vmem, o_vmem):
            @pl.loop(0, BLOCK // GS)
            def _(g):
                base = g * GS
                xs = [x_vmem[pl.ds(base + j * L, L)] for j in range(GS // L)]
                av = jnp.abs(xs[0])
                for j in range(1, GS // L):
                    av = jnp.maximum(av, jnp.abs(xs[j]))
                amax = jnp.max(av)
                amaxv = jnp.full((L,), amax, jnp.float32)
                scale = jnp.where(
                    amaxv == 0.0, INV_MAXQ2, amaxv * INV_MAXQ2
                )
                inv = 1.0 / scale
                for j in range(GS // L):
                    q = xs[j] * inv + zv
                    r = nearest(q)
                    o_vmem[pl.ds(base + j * L, L)] = (r - zv) * scale

        pltpu.emit_pipeline(
            body,
            grid=(n // BLOCK,),
            in_specs=[pl.BlockSpec((BLOCK,), lambda i: (i,))],
            out_specs=[pl.BlockSpec((BLOCK,), lambda i: (i,))],
            core_axis_name=("c", "s"),
            dimension_semantics=(pltpu.PARALLEL,),
        )(x_hbm, o_hbm)

    return run(x1, lookup_values).reshape(shape)
